# Initial kernel scaffold; baseline (speedup 1.0000x reference)
import functools

import jax
import jax.numpy as jnp
from jax import lax
from jax.experimental import pallas as pl
from jax.experimental.pallas import tpu as pltpu
from jax.experimental.pallas import tpu_sc as plsc

N = 10000
D = 128
NPAD = 10240
E = 320000
NC, NS, LANES = 2, 16, 16
NW = NC * NS
CHUNK = 128
EPAD = ((E + NW * CHUNK - 1) // (NW * CHUNK)) * (NW * CHUNK)
EPT = EPAD // NW
NCHUNK = EPT // CHUNK
RPT = NPAD // NS
DEG_R = NPAD // D
DEG_RPT = DEG_R // NS

_mesh = plsc.VectorSubcoreMesh(core_axis_name="c", subcore_axis_name="s")


@functools.partial(
    pl.kernel,
    out_type=jax.ShapeDtypeStruct((NC, DEG_R, D), jnp.float32),
    mesh=_mesh,
    scratch_types=[
        pltpu.VMEM_SHARED((DEG_R, D), jnp.float32),
        pltpu.VMEM((DEG_R, D), jnp.float32),
        pltpu.VMEM((CHUNK,), jnp.int32),
        pltpu.VMEM((DEG_R,), jnp.int32),
    ],
)
def _sc_degree(dst_hbm, out_hbm, deg_sh, acc, idxbuf, rowidx):
    c = lax.axis_index("c")
    s = lax.axis_index("s")
    w = c * NS + s

    zeros16 = jnp.zeros((LANES,), jnp.float32)
    ones16 = jnp.ones((LANES,), jnp.float32)

    def _zrow(r, carry):
        for j in range(D // LANES):
            acc[r, pl.ds(j * LANES, LANES)] = zeros16
        return carry
    lax.fori_loop(0, DEG_R, _zrow, 0)
    pltpu.sync_copy(acc.at[pl.ds(s * DEG_RPT, DEG_RPT)],
                    deg_sh.at[pl.ds(s * DEG_RPT, DEG_RPT)])
    plsc.subcore_barrier()

    def _chunk(i, carry):
        base = w * EPT + i * CHUNK
        pltpu.sync_copy(dst_hbm.at[pl.ds(base, CHUNK)], idxbuf)
        for k in range(CHUNK // LANES):
            d = idxbuf[pl.ds(k * LANES, LANES)]
            row = lax.shift_right_logical(d, 7)
            col = jnp.bitwise_and(d, D - 1)
            plsc.addupdate_scatter(acc, [row, col], ones16)
        return carry
    lax.fori_loop(0, NCHUNK, _chunk, 0)

    for j in range(DEG_R // LANES):
        rowidx[pl.ds(j * LANES, LANES)] = lax.iota(jnp.int32, LANES) + j * LANES
    pltpu.sync_copy(acc, deg_sh.at[rowidx], add=True)
    plsc.subcore_barrier()

    pltpu.sync_copy(deg_sh.at[pl.ds(s * DEG_RPT, DEG_RPT)],
                    out_hbm.at[c, pl.ds(s * DEG_RPT, DEG_RPT)])


@functools.partial(
    pl.kernel,
    out_type=jax.ShapeDtypeStruct((NC, NPAD, D), jnp.float32),
    mesh=_mesh,
    scratch_types=[
        pltpu.VMEM_SHARED((NPAD, D), jnp.float32),
        pltpu.VMEM((LANES, D), jnp.float32),
        pltpu.VMEM((CHUNK,), jnp.int32),
        pltpu.VMEM((CHUNK,), jnp.int32),
        pltpu.VMEM((CHUNK, D), jnp.float32),
        pltpu.SemaphoreType.DMA,
    ],
)
def _sc_aggregate(g_hbm, src_hbm, dst_hbm, out_hbm,
                  acc_sh, zbuf, srcbuf, dstbuf, rows, sem):
    c = lax.axis_index("c")
    s = lax.axis_index("s")
    w = c * NS + s

    zeros16 = jnp.zeros((LANES,), jnp.float32)
    for r in range(LANES):
        for j in range(D // LANES):
            zbuf[r, pl.ds(j * LANES, LANES)] = zeros16

    def _zero(i, carry):
        pltpu.sync_copy(zbuf, acc_sh.at[pl.ds(s * RPT + i * LANES, LANES)])
        return carry
    lax.fori_loop(0, RPT // LANES, _zero, 0)
    plsc.subcore_barrier()

    def _chunk(i, carry):
        base = w * EPT + i * CHUNK
        pltpu.sync_copy(src_hbm.at[pl.ds(base, CHUNK)], srcbuf)
        pltpu.sync_copy(dst_hbm.at[pl.ds(base, CHUNK)], dstbuf)
        pltpu.async_copy(g_hbm.at[srcbuf], rows, sem).wait()
        pltpu.sync_copy(rows, acc_sh.at[dstbuf], add=True)
        return carry
    lax.fori_loop(0, NCHUNK, _chunk, 0)
    plsc.subcore_barrier()

    pltpu.sync_copy(acc_sh.at[pl.ds(s * RPT, RPT)],
                    out_hbm.at[c, pl.ds(s * RPT, RPT)])


def _dinv_body(deg_ref, o_ref):
    deg = deg_ref[0] + deg_ref[1] + 1.0
    node = (lax.broadcasted_iota(jnp.int32, (DEG_R, D), 0) * D
            + lax.broadcasted_iota(jnp.int32, (DEG_R, D), 1))
    dinv = lax.rsqrt(jnp.maximum(deg, 1e-12))
    o_ref[...] = jnp.where(node < N, dinv, 0.0)


def _compute_dinv(deg2):
    return pl.pallas_call(
        _dinv_body,
        out_shape=jax.ShapeDtypeStruct((DEG_R, D), jnp.float32),
        in_specs=[pl.BlockSpec((NC, DEG_R, D), lambda: (0, 0, 0))],
        out_specs=pl.BlockSpec((DEG_R, D), lambda: (0, 0)),
    )(deg2)


_BLK = 1024


def _make_layer_body(combine, relu, matmul):
    def body(*refs):
        refs = list(refs)
        if combine:
            agg, g, dinv, b = refs[:4]
            refs = refs[4:]
            f = (agg[0] + agg[1] + g[...]) * dinv[...] + b[...]
            if relu:
                f = jnp.maximum(f, 0.0)
        else:
            x, dinv = refs[:2]
            refs = refs[2:]
            f = x[...]
        if matmul:
            w_ref, o_ref = refs
            o_ref[...] = jnp.dot(f, w_ref[...],
                                 preferred_element_type=jnp.float32) * dinv[...]
        else:
            (o_ref,) = refs
            o_ref[...] = f
    return body


def _tc_layer(agg, g, dinv, b, w, *, combine, relu, matmul):
    row = pl.BlockSpec((_BLK, D), lambda i: (i, 0))
    in_specs = []
    ins = []
    if combine:
        in_specs += [pl.BlockSpec((NC, _BLK, D), lambda i: (0, i, 0)), row,
                     pl.BlockSpec((_BLK, 1), lambda i: (i, 0)),
                     pl.BlockSpec((1, D), lambda i: (0, 0))]
        ins += [agg, g, dinv, b]
    else:
        in_specs += [row, pl.BlockSpec((_BLK, 1), lambda i: (i, 0))]
        ins += [g, dinv]
    if matmul:
        in_specs += [pl.BlockSpec((D, D), lambda i: (0, 0))]
        ins += [w]
    return pl.pallas_call(
        _make_layer_body(combine, relu, matmul),
        grid=(NPAD // _BLK,),
        out_shape=jax.ShapeDtypeStruct((NPAD, D), jnp.float32),
        in_specs=in_specs,
        out_specs=row,
    )(*ins)


def kernel(x, edge_index0, W_enc, b_enc, W_dec, b_dec,
           W_0, b_0, W_1, b_1, W_2, b_2):
    src = edge_index0[0].astype(jnp.int32)
    dst = edge_index0[1].astype(jnp.int32)
    pad = EPAD - E
    src = jnp.concatenate([src, jnp.full((pad,), N, jnp.int32)])
    dst = jnp.concatenate([dst, jnp.full((pad,), N, jnp.int32)])
    xp = jnp.pad(x, ((0, NPAD - N), (0, 0)))

    deg2 = _sc_degree(dst)
    dinv = _compute_dinv(deg2).reshape(NPAD, 1)

    g = _tc_layer(None, xp, dinv, None, W_enc,
                  combine=False, relu=False, matmul=True)

    steps = [(b_enc, W_0, False), (b_0, W_1, True),
             (b_1, W_2, True), (b_2, W_dec, True)]
    for b_prev, w_next, relu in steps:
        agg = _sc_aggregate(g, src, dst)
        g = _tc_layer(agg, g, dinv, b_prev.reshape(1, D), w_next,
                      combine=True, relu=relu, matmul=True)

    agg = _sc_aggregate(g, src, dst)
    out = _tc_layer(agg, g, dinv, b_dec.reshape(1, D), None,
                    combine=True, relu=False, matmul=False)
    return out[:N]


# speedup vs baseline: 7.4905x; 7.4905x over previous
import functools

import jax
import jax.numpy as jnp
from jax import lax
from jax.experimental import pallas as pl
from jax.experimental.pallas import tpu as pltpu
from jax.experimental.pallas import tpu_sc as plsc

N = 10000
D = 128
NPAD = 10240
E = 320000
NC, NS, LANES = 2, 16, 16
NW = NC * NS
CHUNK = 128
EPAD = ((E + NW * CHUNK - 1) // (NW * CHUNK)) * (NW * CHUNK)
EPT = EPAD // NW
NCHUNK = EPT // CHUNK
RPT = NPAD // NS
DEG_R = NPAD // D
DEG_RPT = DEG_R // NS

_mesh = plsc.VectorSubcoreMesh(core_axis_name="c", subcore_axis_name="s")


@functools.partial(
    pl.kernel,
    out_type=jax.ShapeDtypeStruct((NC * NPAD,), jnp.float32),
    mesh=_mesh,
    scratch_types=[
        pltpu.VMEM_SHARED((NPAD,), jnp.float32),
        pltpu.VMEM((RPT,), jnp.float32),
        pltpu.VMEM((CHUNK,), jnp.float32),
        pltpu.VMEM((CHUNK,), jnp.int32),
    ],
)
def _sc_degree(dst_hbm, out_hbm, deg_sh, zbuf, ones, idxbuf):
    c = lax.axis_index("c")
    s = lax.axis_index("s")
    w = c * NS + s

    zeros16 = jnp.zeros((LANES,), jnp.float32)
    ones16 = jnp.ones((LANES,), jnp.float32)

    def _z(i, carry):
        zbuf[pl.ds(i * LANES, LANES)] = zeros16
        return carry
    lax.fori_loop(0, RPT // LANES, _z, 0)
    for j in range(CHUNK // LANES):
        ones[pl.ds(j * LANES, LANES)] = ones16
    pltpu.sync_copy(zbuf, deg_sh.at[pl.ds(s * RPT, RPT)])
    plsc.subcore_barrier()

    def _chunk(i, carry):
        base = w * EPT + i * CHUNK
        pltpu.sync_copy(dst_hbm.at[pl.ds(base, CHUNK)], idxbuf)
        pltpu.sync_copy(ones, deg_sh.at[idxbuf], add=True)
        return carry
    lax.fori_loop(0, NCHUNK, _chunk, 0)
    plsc.subcore_barrier()

    pltpu.sync_copy(deg_sh.at[pl.ds(s * RPT, RPT)],
                    out_hbm.at[pl.ds(c * NPAD + s * RPT, RPT)])


@functools.partial(
    pl.kernel,
    out_type=jax.ShapeDtypeStruct((NC, NPAD, D), jnp.float32),
    mesh=_mesh,
    scratch_types=[
        pltpu.VMEM_SHARED((NPAD, D), jnp.float32),
        pltpu.VMEM((LANES, D), jnp.float32),
        pltpu.VMEM((CHUNK,), jnp.int32),
        pltpu.VMEM((CHUNK,), jnp.int32),
        pltpu.VMEM((CHUNK, D), jnp.float32),
        pltpu.SemaphoreType.DMA,
    ],
)
def _sc_aggregate(g_hbm, src_hbm, dst_hbm, out_hbm,
                  acc_sh, zbuf, srcbuf, dstbuf, rows, sem):
    c = lax.axis_index("c")
    s = lax.axis_index("s")
    w = c * NS + s

    zeros16 = jnp.zeros((LANES,), jnp.float32)
    for r in range(LANES):
        for j in range(D // LANES):
            zbuf[r, pl.ds(j * LANES, LANES)] = zeros16

    def _zero(i, carry):
        pltpu.sync_copy(zbuf, acc_sh.at[pl.ds(s * RPT + i * LANES, LANES)])
        return carry
    lax.fori_loop(0, RPT // LANES, _zero, 0)
    plsc.subcore_barrier()

    def _chunk(i, carry):
        base = w * EPT + i * CHUNK
        pltpu.sync_copy(src_hbm.at[pl.ds(base, CHUNK)], srcbuf)
        pltpu.sync_copy(dst_hbm.at[pl.ds(base, CHUNK)], dstbuf)
        pltpu.async_copy(g_hbm.at[srcbuf], rows, sem).wait()
        pltpu.sync_copy(rows, acc_sh.at[dstbuf], add=True)
        return carry
    lax.fori_loop(0, NCHUNK, _chunk, 0)
    plsc.subcore_barrier()

    pltpu.sync_copy(acc_sh.at[pl.ds(s * RPT, RPT)],
                    out_hbm.at[c, pl.ds(s * RPT, RPT)])


def _dinv_body(deg_ref, o_ref):
    deg = jnp.sum(deg_ref[...], axis=0) + 1.0
    node = (lax.broadcasted_iota(jnp.int32, (DEG_R, D), 0) * D
            + lax.broadcasted_iota(jnp.int32, (DEG_R, D), 1))
    dinv = lax.rsqrt(jnp.maximum(deg, 1e-12))
    o_ref[...] = jnp.where(node < N, dinv, 0.0)


def _compute_dinv(degp):
    return pl.pallas_call(
        _dinv_body,
        out_shape=jax.ShapeDtypeStruct((DEG_R, D), jnp.float32),
        in_specs=[pl.BlockSpec((NC, DEG_R, D), lambda: (0, 0, 0))],
        out_specs=pl.BlockSpec((DEG_R, D), lambda: (0, 0)),
    )(degp)


_BLK = 1024


def _make_layer_body(combine, relu, matmul):
    def body(*refs):
        refs = list(refs)
        if combine:
            agg, g, dinv, b = refs[:4]
            refs = refs[4:]
            f = (agg[0] + agg[1] + g[...]) * dinv[...] + b[...]
            if relu:
                f = jnp.maximum(f, 0.0)
        else:
            x, dinv = refs[:2]
            refs = refs[2:]
            f = x[...]
        if matmul:
            w_ref, o_ref = refs
            o_ref[...] = jnp.dot(f, w_ref[...],
                                 preferred_element_type=jnp.float32) * dinv[...]
        else:
            (o_ref,) = refs
            o_ref[...] = f
    return body


def _tc_layer(agg, g, dinv, b, w, *, combine, relu, matmul):
    row = pl.BlockSpec((_BLK, D), lambda i: (i, 0))
    in_specs = []
    ins = []
    if combine:
        in_specs += [pl.BlockSpec((NC, _BLK, D), lambda i: (0, i, 0)), row,
                     pl.BlockSpec((_BLK, 1), lambda i: (i, 0)),
                     pl.BlockSpec((1, D), lambda i: (0, 0))]
        ins += [agg, g, dinv, b]
    else:
        in_specs += [row, pl.BlockSpec((_BLK, 1), lambda i: (i, 0))]
        ins += [g, dinv]
    if matmul:
        in_specs += [pl.BlockSpec((D, D), lambda i: (0, 0))]
        ins += [w]
    return pl.pallas_call(
        _make_layer_body(combine, relu, matmul),
        grid=(NPAD // _BLK,),
        out_shape=jax.ShapeDtypeStruct((NPAD, D), jnp.float32),
        in_specs=in_specs,
        out_specs=row,
    )(*ins)


def kernel(x, edge_index0, W_enc, b_enc, W_dec, b_dec,
           W_0, b_0, W_1, b_1, W_2, b_2):
    src = edge_index0[0].astype(jnp.int32)
    dst = edge_index0[1].astype(jnp.int32)
    pad = EPAD - E
    src = jnp.concatenate([src, jnp.full((pad,), N, jnp.int32)])
    dst = jnp.concatenate([dst, jnp.full((pad,), N, jnp.int32)])
    xp = jnp.pad(x, ((0, NPAD - N), (0, 0)))

    degp = _sc_degree(dst).reshape(NC, DEG_R, D)
    dinv = _compute_dinv(degp).reshape(NPAD, 1)

    g = _tc_layer(None, xp, dinv, None, W_enc,
                  combine=False, relu=False, matmul=True)

    steps = [(b_enc, W_0, False), (b_0, W_1, True),
             (b_1, W_2, True), (b_2, W_dec, True)]
    for b_prev, w_next, relu in steps:
        agg = _sc_aggregate(g, src, dst)
        g = _tc_layer(agg, g, dinv, b_prev.reshape(1, D), w_next,
                      combine=True, relu=relu, matmul=True)

    agg = _sc_aggregate(g, src, dst)
    out = _tc_layer(agg, g, dinv, b_dec.reshape(1, D), None,
                    combine=True, relu=False, matmul=False)
    return out[:N]
